# baseline (speedup 1.0000x reference)
import functools

import jax
import jax.numpy as jnp
from jax import lax
from jax.experimental import pallas as pl
from jax.experimental.pallas import tpu as pltpu
from jax.experimental.pallas import tpu_sc as plsc

VOCAB = 100002
EMB = 300
BATCH = 16384
NUM_CORES = 2
NUM_SUBCORES = 16
NUM_WORKERS = NUM_CORES * NUM_SUBCORES
MAX_ROWS = -(-EMB // NUM_WORKERS)
LANES = 16
TAIL_ROWS = 40
MAIN = 99968
SEG = 33408
SEG_LEN = (SEG, SEG, MAIN - 2 * SEG)
TAIL_LO = VOCAB - TAIL_ROWS
NVEC = BATCH // LANES
LIST_CAP = BATCH + 5 * LANES


def _build():
    mesh = plsc.VectorSubcoreMesh(core_axis_name="c", subcore_axis_name="s")

    @functools.partial(
        pl.kernel,
        mesh=mesh,
        compiler_params=pltpu.CompilerParams(needs_layout_passes=False),
        out_type=jax.ShapeDtypeStruct((EMB, BATCH), jnp.float32),
        scratch_types=[
            pltpu.VMEM((SEG,), jnp.float32),
            pltpu.VMEM((SEG,), jnp.float32),
            pltpu.VMEM((LIST_CAP,), jnp.int32),
            pltpu.VMEM((BATCH + LANES,), jnp.float32),
            pltpu.VMEM((BATCH + LANES,), jnp.float32),
            pltpu.VMEM((EMB * TAIL_ROWS,), jnp.float32),
            pltpu.SemaphoreType.DMA,
            pltpu.SemaphoreType.DMA,
        ],
    )
    def emb_kernel(words_f_hbm, table_t_hbm, tail_t_hbm, out_t_hbm,
                   seg_a, seg_b, encl, out_a, out_b, tail_v, ssem, osem):
        wid = lax.axis_index("s") * NUM_CORES + lax.axis_index("c")
        n_rows = jnp.where(wid < EMB - (MAX_ROWS - 1) * NUM_WORKERS,
                           MAX_ROWS, MAX_ROWS - 1)
        rings = (seg_a, seg_b)

        def stream_seg(d, s, p):
            pltpu.async_copy(
                table_t_hbm.at[d, pl.ds(s * SEG, SEG_LEN[s])],
                rings[p].at[pl.ds(0, SEG_LEN[s])], ssem)

        def drain_seg(s):
            pltpu.make_async_copy(
                table_t_hbm.at[0, pl.ds(0, SEG_LEN[s])],
                seg_a.at[pl.ds(0, SEG_LEN[s])], ssem).wait()

        pltpu.sync_copy(words_f_hbm, seg_b.at[pl.ds(0, BATCH)])
        stream_seg(wid, 0, 0)
        pltpu.sync_copy(tail_t_hbm, tail_v)

        iota = lax.iota(jnp.int32, LANES)
        zeros = jnp.zeros((LANES,), jnp.int32)

        def ceil_lanes(n):
            return ((n + LANES - 1) // LANES) * LANES

        def count_body(i, carry):
            c0, c01, c02 = carry
            vec = plsc.bitcast(seg_b[pl.ds(i * LANES, LANES)], jnp.int32)
            c0 = c0 + plsc.all_reduce_population_count(vec < SEG)
            c01 = c01 + plsc.all_reduce_population_count(vec < 2 * SEG)
            c02 = c02 + plsc.all_reduce_population_count(vec < TAIL_LO)
            return c0, c01, c02

        c0, c01, c02 = lax.fori_loop(0, NVEC, count_body,
                                     (zeros, zeros, zeros), unroll=False)
        n0 = c0[0]
        n1 = c01[0] - n0
        n2 = c02[0] - c01[0]
        n3 = BATCH - c02[0]
        r1 = ceil_lanes(n0)
        r2 = r1 + ceil_lanes(n1)
        r3 = r2 + ceil_lanes(n2)

        dummy = jnp.full((LANES,), BATCH * 65536, jnp.int32)
        encl[pl.ds(n0, LANES)] = dummy
        encl[pl.ds(r1 + n1, LANES)] = dummy
        encl[pl.ds(r2 + n2, LANES)] = dummy
        encl[pl.ds(r3 + n3, LANES)] = dummy

        def part_body(i, carry):
            o0, o1, o2, o3 = carry
            vec = plsc.bitcast(seg_b[pl.ds(i * LANES, LANES)], jnp.int32)
            pos16 = (iota + i * LANES) * 65536
            m0 = vec < SEG
            m1 = jnp.logical_and(vec >= SEG, vec < 2 * SEG)
            m3 = vec >= TAIL_LO
            m2 = jnp.logical_and(vec >= 2 * SEG, vec < TAIL_LO)
            plsc.store_compressed(encl.at[pl.ds(o0, LANES)],
                                  pos16 | vec, mask=m0)
            plsc.store_compressed(encl.at[pl.ds(o1, LANES)],
                                  pos16 | (vec - SEG), mask=m1)
            plsc.store_compressed(encl.at[pl.ds(o2, LANES)],
                                  pos16 | (vec - 2 * SEG), mask=m2)
            plsc.store_compressed(encl.at[pl.ds(o3, LANES)],
                                  pos16 | (vec - TAIL_LO), mask=m3)
            o0 = o0 + plsc.all_reduce_population_count(m0)[0]
            o1 = o1 + plsc.all_reduce_population_count(m1)[0]
            o2 = o2 + plsc.all_reduce_population_count(m2)[0]
            o3 = o3 + plsc.all_reduce_population_count(m3)[0]
            return o0, o1, o2, o3

        lax.fori_loop(0, NVEC, part_body, (jnp.int32(0), r1, r2, r3),
                      unroll=False)

        starts = (jnp.int32(0), r1, r2, r3)
        counts = (ceil_lanes(n0) // LANES, ceil_lanes(n1) // LANES,
                  ceil_lanes(n2) // LANES, ceil_lanes(n3) // LANES)

        def gather_seg(s, p, out_ref):
            rs = starts[s]
            buf = rings[p]

            def gbody(v, carry):
                enc = encl[pl.ds(rs + v * LANES, LANES)]
                iv = jnp.bitwise_and(enc, 65535)
                pv = lax.shift_right_logical(enc, 16)
                plsc.store_scatter(out_ref, [pv], plsc.load_gather(buf, [iv]))
                return carry

            lax.fori_loop(0, counts[s], gbody, 0, unroll=False)

        def gather_tail(d, out_ref):
            rs = starts[3]
            base = d * TAIL_ROWS

            def gbody(v, carry):
                enc = encl[pl.ds(rs + v * LANES, LANES)]
                iv = jnp.bitwise_and(enc, 65535) + base
                pv = lax.shift_right_logical(enc, 16)
                plsc.store_scatter(out_ref, [pv],
                                   plsc.load_gather(tail_v, [iv]))
                return carry

            lax.fori_loop(0, counts[3], gbody, 0, unroll=False)

        def flush(out_ref, d):
            pltpu.async_copy(out_ref.at[pl.ds(0, BATCH)],
                             out_t_hbm.at[d], osem).wait()

        n_iter = (n_rows + 1) // 2

        def pair_body(t, carry):
            d_a = wid + (2 * t) * NUM_WORKERS
            d_b = d_a + NUM_WORKERS
            has_b = (2 * t + 1) < n_rows

            drain_seg(0)
            stream_seg(d_a, 1, 1)
            gather_tail(d_a, out_a)
            gather_seg(0, 0, out_a)

            drain_seg(1)
            stream_seg(d_a, 2, 0)
            gather_seg(1, 1, out_a)

            drain_seg(2)

            @pl.when(has_b)
            def _():
                stream_seg(d_b, 0, 1)

            gather_seg(2, 0, out_a)
            flush(out_a, d_a)

            @pl.when(has_b)
            def _():
                drain_seg(0)
                stream_seg(d_b, 1, 0)
                gather_tail(d_b, out_b)
                gather_seg(0, 1, out_b)

                drain_seg(1)
                stream_seg(d_b, 2, 1)
                gather_seg(1, 0, out_b)

                drain_seg(2)

                @pl.when(t + 1 < n_iter)
                def _():
                    stream_seg(d_a + 2 * NUM_WORKERS, 0, 0)

                gather_seg(2, 1, out_b)
                flush(out_b, d_b)

            return carry

        lax.fori_loop(0, n_iter, pair_body, 0, unroll=False)

    return emb_kernel


_emb_lookup = _build()


def kernel(words, table):
    words_f = lax.bitcast_convert_type(words.astype(jnp.int32), jnp.float32)
    tail_t = jnp.reshape(jnp.transpose(table[VOCAB - TAIL_ROWS:]), (-1,))
    out_t = _emb_lookup(words_f, table.T, tail_t)
    return out_t.T
